# Initial kernel scaffold; baseline (speedup 1.0000x reference)
import jax
import jax.numpy as jnp
from jax.experimental import pallas as pl


def kernel(positive_triplets, negative_triplets, rel_weight):
    raise NotImplementedError("write your pallas kernel here")


# speedup vs baseline: 1.7137x; 1.7137x over previous
import functools

import jax
import jax.numpy as jnp
from jax import lax
from jax.experimental import pallas as pl
from jax.experimental.pallas import tpu as pltpu
from jax.experimental.pallas import tpu_sc as plsc

_B = 16384
_DIM = 128
_NC = 2
_NS = 16
_NW = _NC * _NS
_RPW = _B // _NW
_L = 16
_G = _DIM // _L

_MESH = plsc.VectorSubcoreMesh(
    core_axis_name="c", subcore_axis_name="s", num_cores=_NC, num_subcores=_NS
)


@functools.partial(
    pl.kernel,
    out_type=[jax.ShapeDtypeStruct((_B,), jnp.float32)] * 3,
    mesh=_MESH,
    compiler_params=pltpu.CompilerParams(needs_layout_passes=False),
    scratch_types=[
        pltpu.VMEM((_RPW + 8, _DIM), jnp.int32),
        pltpu.VMEM((8, _DIM), jnp.float32),
        pltpu.VMEM((_RPW,), jnp.float32),
        pltpu.VMEM((_RPW,), jnp.float32),
        pltpu.VMEM((_RPW,), jnp.float32),
    ],
)
def _transe_sc(pos_hbm, neg_hbm, rel_hbm, loss_o, pos_o, neg_o,
               buf, r0_v, pos_v, neg_v, loss_v):
    wid = lax.axis_index("c") * _NS + lax.axis_index("s")
    base = wid * _RPW

    pltpu.sync_copy(rel_hbm.at[pl.ds(0, 8)], r0_v)
    r0 = [r0_v[0, pl.ds(g * _L, _L)] for g in range(_G)]

    def load_slice(src_hbm):
        @pl.when(wid < _NW - 1)
        def _():
            pltpu.sync_copy(src_hbm.at[pl.ds(base, _RPW + 8)], buf)

        @pl.when(wid == _NW - 1)
        def _():
            pltpu.sync_copy(src_hbm.at[pl.ds(base, _RPW)],
                            buf.at[pl.ds(0, _RPW)])

    lanes = lax.iota(jnp.int32, _L)
    masks = [lanes == r for r in range(_L)]

    def compute(out_v, head_off, tail_off):
        def grp(gi, carry):
            rb = gi * _L
            s_vec = jnp.zeros((_L,), jnp.float32)
            for r in range(_L):
                i = rb + r
                acc = None
                for g in range(_G):
                    a = buf[i + head_off, pl.ds(g * _L, _L)]
                    b = buf[i + tail_off, pl.ds(g * _L, _L)]
                    d = (a - b).astype(jnp.float32) + r0[g]
                    acc = jnp.abs(d) if acc is None else acc + jnp.abs(d)
                s_vec = jnp.where(masks[r], jnp.sum(acc), s_vec)
            out_v[pl.ds(rb, _L)] = s_vec
            return carry

        lax.fori_loop(0, _RPW // _L, grp, 0)

    load_slice(pos_hbm)
    compute(pos_v, 0, 1)
    load_slice(neg_hbm)
    compute(neg_v, 1, 0)

    for c in range(_RPW // _L):
        p = pos_v[pl.ds(c * _L, _L)]
        q = neg_v[pl.ds(c * _L, _L)]
        loss_v[pl.ds(c * _L, _L)] = jnp.maximum(p - q + 1.0, 0.0)

    pltpu.sync_copy(loss_v, loss_o.at[pl.ds(base, _RPW)])
    pltpu.sync_copy(pos_v, pos_o.at[pl.ds(base, _RPW)])
    pltpu.sync_copy(neg_v, neg_o.at[pl.ds(base, _RPW)])


def kernel(positive_triplets, negative_triplets, rel_weight):
    loss, pos, neg = _transe_sc(positive_triplets, negative_triplets,
                                rel_weight)
    return loss[: _B - 1], pos[: _B - 1], neg[: _B - 1]


# speedup vs baseline: 1.9222x; 1.1217x over previous
import functools

import jax
import jax.numpy as jnp
from jax import lax
from jax.experimental import pallas as pl
from jax.experimental.pallas import tpu as pltpu
from jax.experimental.pallas import tpu_sc as plsc

_B = 16384
_DIM = 128
_NC = 2
_NS = 16
_NW = _NC * _NS
_RPW = _B // _NW
_L = 16
_G = _DIM // _L
_CH = 256
_CR = _CH + 8

_MESH = plsc.VectorSubcoreMesh(
    core_axis_name="c", subcore_axis_name="s", num_cores=_NC, num_subcores=_NS
)


@functools.partial(
    pl.kernel,
    out_type=[jax.ShapeDtypeStruct((_B,), jnp.float32)] * 3,
    mesh=_MESH,
    compiler_params=pltpu.CompilerParams(needs_layout_passes=False),
    scratch_types=[
        pltpu.VMEM((_CR + 8, _DIM), jnp.int32),
        pltpu.VMEM((_CR + 8, _DIM), jnp.int32),
        pltpu.VMEM((8, _DIM), jnp.float32),
        pltpu.VMEM((_RPW,), jnp.float32),
        pltpu.VMEM((_RPW,), jnp.float32),
        pltpu.VMEM((_RPW,), jnp.float32),
        pltpu.SemaphoreType.DMA,
        pltpu.SemaphoreType.DMA,
    ],
)
def _transe_sc(pos_hbm, neg_hbm, rel_hbm, loss_o, pos_o, neg_o,
               buf_a, buf_b, r0_v, pos_v, neg_v, loss_v, sem_a, sem_b):
    wid = lax.axis_index("c") * _NS + lax.axis_index("s")
    base = wid * _RPW

    pltpu.sync_copy(rel_hbm.at[pl.ds(0, 8)], r0_v)
    r0 = [r0_v[0, pl.ds(g * _L, _L)] for g in range(_G)]

    lanes = lax.iota(jnp.int32, _L)
    masks = [lanes == r for r in range(_L)]

    bufs = (buf_a, buf_b)
    sems = (sem_a, sem_b)

    descs = (
        (pos_hbm, pos_v, 0, False),
        (pos_hbm, pos_v, 1, False),
        (neg_hbm, neg_v, 0, True),
        (neg_hbm, neg_v, 1, True),
    )

    def issue(idx):
        src = descs[idx][0]
        su = base + descs[idx][2] * _CH
        start = jnp.minimum(su, _B - _CR)
        return pltpu.async_copy(
            src.at[pl.ds(start, _CR)], bufs[idx % 2].at[pl.ds(0, _CR)],
            sems[idx % 2]), su - start

    def compute(idx, off):
        buf = bufs[idx % 2]
        out_v = descs[idx][1]
        k0 = descs[idx][2] * _CH
        swap = descs[idx][3]

        def load_row(k):
            return [buf[k + off, pl.ds(g * _L, _L)] for g in range(_G)]

        def grp(gi, prev):
            rb = gi * _L
            s_vec = jnp.zeros((_L,), jnp.float32)
            for r in range(_L):
                new = load_row(rb + r + 1)
                acc = None
                for g in range(_G):
                    dint = new[g] - prev[g] if swap else prev[g] - new[g]
                    d = dint.astype(jnp.float32) + r0[g]
                    acc = jnp.abs(d) if acc is None else acc + jnp.abs(d)
                s_vec = jnp.where(masks[r], jnp.sum(acc), s_vec)
                prev = new
            out_v[pl.ds(k0 + rb, _L)] = s_vec
            return tuple(prev)

        lax.fori_loop(0, _CH // _L, grp, tuple(load_row(0)))

    cps = [issue(0), issue(1)]
    for idx in range(4):
        cps[idx][0].wait()
        compute(idx, cps[idx][1])
        if idx + 2 < 4:
            cps.append(issue(idx + 2))

    for c in range(_RPW // _L):
        p = pos_v[pl.ds(c * _L, _L)]
        q = neg_v[pl.ds(c * _L, _L)]
        loss_v[pl.ds(c * _L, _L)] = jnp.maximum(p - q + 1.0, 0.0)

    pltpu.sync_copy(loss_v, loss_o.at[pl.ds(base, _RPW)])
    pltpu.sync_copy(pos_v, pos_o.at[pl.ds(base, _RPW)])
    pltpu.sync_copy(neg_v, neg_o.at[pl.ds(base, _RPW)])


def kernel(positive_triplets, negative_triplets, rel_weight):
    loss, pos, neg = _transe_sc(positive_triplets, negative_triplets,
                                rel_weight)
    return loss[: _B - 1], pos[: _B - 1], neg[: _B - 1]
